# baseline (speedup 1.0000x reference)
import functools

import jax
import jax.numpy as jnp
from jax import lax
from jax.experimental import pallas as pl
from jax.experimental.pallas import tpu as pltpu

_B = 1024
_V = 100000
_K = 8
_G = _B // _K


def _index_map(g, trg_ref, *, k):
  return g, lax.shift_right_logical(trg_ref[_K * g + k], 7)


def _ce_body(trg_ref, *refs):
  blocks, out_ref = refs[:_K], refs[_K]
  g = pl.program_id(0)
  sub = lax.broadcasted_iota(jnp.int32, (_K, 128), 0)
  lane = lax.broadcasted_iota(jnp.int32, (_K, 128), 1)
  filled = jnp.ones((_K, 128), jnp.float32)
  for k in range(_K):
    t = trg_ref[_K * g + k]
    filled = jnp.where((sub == k) & (lane == (t & 127)), blocks[k][...], filled)
  step_loss = jnp.sum(-jnp.log(filled))

  @pl.when(g == 0)
  def _():
    out_ref[0, 0] = 0.0

  out_ref[0, 0] += step_loss


@jax.jit
def kernel(prd, trg):
  grid_spec = pltpu.PrefetchScalarGridSpec(
      num_scalar_prefetch=1,
      grid=(_G,),
      in_specs=[
          pl.BlockSpec((_K, 128), functools.partial(_index_map, k=k))
          for k in range(_K)
      ],
      out_specs=pl.BlockSpec(
          (1, 1), lambda g, trg_ref: (0, 0), memory_space=pltpu.SMEM
      ),
  )
  out = pl.pallas_call(
      _ce_body,
      grid_spec=grid_spec,
      out_shape=jax.ShapeDtypeStruct((1, 1), jnp.float32),
  )(trg.astype(jnp.int32), *([prd] * _K))
  return out[0, 0]


# speedup vs baseline: 1.1345x; 1.1345x over previous
import jax
import jax.numpy as jnp
from jax import lax
from jax.experimental import pallas as pl
from jax.experimental.pallas import tpu as pltpu

_B = 1024
_V = 100000
_SPG = 16
_G = _B // _SPG
_NBUF = 4
_LOOKAHEAD = _NBUF - 1


def _ce_body(trg_ref, prd_ref, out_ref, buf, acc_ref, sems):
  g = pl.program_id(0)

  def tile_copy(step, j):
    i = jnp.minimum(step * _SPG + j, _B - 1)
    t = trg_ref[i]
    colb = pl.multiple_of(lax.shift_left(lax.shift_right_logical(t, 7), 7), 128)
    rb = pl.multiple_of(step * _SPG + (j // 8) * 8, 8)
    return pltpu.make_async_copy(
        prd_ref.at[pl.ds(rb, 8), pl.ds(colb, 128)],
        buf.at[step % _NBUF, j],
        sems.at[step % _NBUF, j],
    )

  @pl.when(g == 0)
  def _():
    acc_ref[...] = jnp.zeros((_SPG, 128), jnp.float32)
    for step in range(_LOOKAHEAD):
      for j in range(_SPG):
        tile_copy(step, j).start()

  @pl.when(g < _G - _LOOKAHEAD)
  def _():
    for j in range(_SPG):
      tile_copy(g + _LOOKAHEAD, j).start()

  lane = lax.broadcasted_iota(jnp.int32, (8, 128), 1)
  sub = lax.broadcasted_iota(jnp.int32, (8, 128), 0)
  for h in range(_SPG // 8):
    filled = jnp.ones((8, 128), jnp.float32)
    for k in range(8):
      j = h * 8 + k
      tile_copy(g, j).wait()
      t = trg_ref[g * _SPG + j]
      filled = jnp.where(
          (sub == k) & (lane == (t & 127)),
          buf[(g % _NBUF).astype(jnp.int32), j],
          filled,
      )
    acc_ref[pl.ds(h * 8, 8), :] += jnp.log(filled)

  @pl.when(g == _G - 1)
  def _():
    out_ref[0, 0] = -jnp.sum(acc_ref[...])


@jax.jit
def kernel(prd, trg):
  grid_spec = pltpu.PrefetchScalarGridSpec(
      num_scalar_prefetch=1,
      grid=(_G,),
      in_specs=[pl.BlockSpec(memory_space=pltpu.HBM)],
      out_specs=pl.BlockSpec(
          (1, 1), lambda g, trg_ref: (0, 0), memory_space=pltpu.SMEM
      ),
      scratch_shapes=[
          pltpu.VMEM((_NBUF, _SPG, 8, 128), jnp.float32),
          pltpu.VMEM((_SPG, 128), jnp.float32),
          pltpu.SemaphoreType.DMA((_NBUF, _SPG)),
      ],
  )
  out = pl.pallas_call(
      _ce_body,
      grid_spec=grid_spec,
      out_shape=jax.ShapeDtypeStruct((1, 1), jnp.float32),
  )(trg.astype(jnp.int32), prd)
  return out[0, 0]


# speedup vs baseline: 22.2802x; 19.6387x over previous
import jax
import jax.numpy as jnp
from jax import lax
from jax.experimental import pallas as pl
from jax.experimental.pallas import tpu as pltpu

_B = 1024
_V = 100000
_SPG = 16
_G = _B // _SPG
_NBUF = 4
_LOOKAHEAD = _NBUF - 1


def _ce_body(trg_ref, prdt_ref, out_ref, buf, acc_ref, sems):
  g = pl.program_id(0)

  def tile_copy(step, j):
    i = jnp.minimum(step * _SPG + j, _B - 1)
    t = trg_ref[i]
    rb = pl.multiple_of(lax.shift_left(lax.shift_right_logical(t, 3), 3), 8)
    colb = pl.multiple_of((step // 8) * 128, 128)
    return pltpu.make_async_copy(
        prdt_ref.at[pl.ds(rb, 8), pl.ds(colb, 128)],
        buf.at[step % _NBUF, j],
        sems.at[step % _NBUF, j],
    )

  @pl.when(g == 0)
  def _():
    acc_ref[...] = jnp.zeros((_SPG, 128), jnp.float32)
    for step in range(_LOOKAHEAD):
      for j in range(_SPG):
        tile_copy(step, j).start()

  @pl.when(g < _G - _LOOKAHEAD)
  def _():
    for j in range(_SPG):
      tile_copy(g + _LOOKAHEAD, j).start()

  lane = lax.broadcasted_iota(jnp.int32, (8, 128), 1)
  sub = lax.broadcasted_iota(jnp.int32, (8, 128), 0)
  for h in range(_SPG // 8):
    filled = jnp.ones((8, 128), jnp.float32)
    for k in range(8):
      j = h * 8 + k
      tile_copy(g, j).wait()
      t = trg_ref[g * _SPG + j]
      col = (g % 8) * _SPG + j
      filled = jnp.where(
          (sub == (t & 7)) & (lane == col),
          buf[(g % _NBUF).astype(jnp.int32), j],
          filled,
      )
    acc_ref[pl.ds(h * 8, 8), :] += jnp.log(filled)

  @pl.when(g == _G - 1)
  def _():
    out_ref[0, 0] = -jnp.sum(acc_ref[...])


@jax.jit
def kernel(prd, trg):
  grid_spec = pltpu.PrefetchScalarGridSpec(
      num_scalar_prefetch=1,
      grid=(_G,),
      in_specs=[pl.BlockSpec(memory_space=pltpu.HBM)],
      out_specs=pl.BlockSpec(
          (1, 1), lambda g, trg_ref: (0, 0), memory_space=pltpu.SMEM
      ),
      scratch_shapes=[
          pltpu.VMEM((_NBUF, _SPG, 8, 128), jnp.float32),
          pltpu.VMEM((_SPG, 128), jnp.float32),
          pltpu.SemaphoreType.DMA((_NBUF, _SPG)),
      ],
  )
  out = pl.pallas_call(
      _ce_body,
      grid_spec=grid_spec,
      out_shape=jax.ShapeDtypeStruct((1, 1), jnp.float32),
  )(trg.astype(jnp.int32), prd.T)
  return out[0, 0]


# speedup vs baseline: 34.4550x; 1.5464x over previous
import jax
import jax.numpy as jnp
from jax import lax
from jax.experimental import pallas as pl
from jax.experimental.pallas import tpu as pltpu

_B = 1024
_V = 100000
_SPG = 128
_G = _B // _SPG
_NBUF = 2
_SEMGRP = 16


def _ce_body(trg_ref, prdt_ref, out_ref, buf, acc_ref, sems):
  g = pl.program_id(0)

  def tile_copy(step, j):
    t = trg_ref[step * _SPG + j]
    rb = pl.multiple_of(t & ~7, 8)
    colb = pl.multiple_of(step * _SPG, 128)
    return pltpu.make_async_copy(
        prdt_ref.at[pl.ds(rb, 8), pl.ds(colb, 128)],
        buf.at[step % _NBUF, j],
        sems.at[step % _NBUF, j // _SEMGRP],
    )

  def tile_wait(step, j):
    pltpu.make_async_copy(
        prdt_ref.at[pl.ds(0, 8), pl.ds(0, 128)],
        buf.at[step % _NBUF, j],
        sems.at[step % _NBUF, j // _SEMGRP],
    ).wait()

  @pl.when(g == 0)
  def _():
    acc_ref[...] = jnp.zeros((8, 128), jnp.float32)
    for j in range(_SPG):
      tile_copy(0, j).start()

  @pl.when(g < _G - 1)
  def _():
    for j in range(_SPG):
      tile_copy(g + 1, j).start()

  lane = lax.broadcasted_iota(jnp.int32, (8, 128), 1)
  sub = lax.broadcasted_iota(jnp.int32, (8, 128), 0)
  filled = jnp.ones((8, 128), jnp.float32)
  gb = (g % _NBUF).astype(jnp.int32)
  for j in range(_SPG):
    tile_wait(g, j)
    t = trg_ref[g * _SPG + j]
    filled = jnp.where((sub == (t & 7)) & (lane == j), buf[gb, j], filled)
  acc_ref[...] += jnp.log(filled)

  @pl.when(g == _G - 1)
  def _():
    out_ref[0, 0] = -jnp.sum(acc_ref[...])


@jax.jit
def kernel(prd, trg):
  grid_spec = pltpu.PrefetchScalarGridSpec(
      num_scalar_prefetch=1,
      grid=(_G,),
      in_specs=[pl.BlockSpec(memory_space=pltpu.HBM)],
      out_specs=pl.BlockSpec(
          (1, 1), lambda g, trg_ref: (0, 0), memory_space=pltpu.SMEM
      ),
      scratch_shapes=[
          pltpu.VMEM((_NBUF, _SPG, 8, 128), jnp.float32),
          pltpu.VMEM((8, 128), jnp.float32),
          pltpu.SemaphoreType.DMA((_NBUF, _SPG // _SEMGRP)),
      ],
  )
  out = pl.pallas_call(
      _ce_body,
      grid_spec=grid_spec,
      out_shape=jax.ShapeDtypeStruct((1, 1), jnp.float32),
  )(trg.astype(jnp.int32), prd.T)
  return out[0, 0]


# speedup vs baseline: 51.7367x; 1.5016x over previous
import jax
import jax.numpy as jnp
from jax import lax
from jax.experimental import pallas as pl
from jax.experimental.pallas import tpu as pltpu

_B = 1024
_V = 100000
_SPG = 128
_G = _B // _SPG
_NBUF = 2


def _ce_body(trg_ref, prdt_ref, out_ref, buf, acc_ref, sems):
  g = pl.program_id(0)

  def tile_copy(step, j):
    t = trg_ref[step * _SPG + j]
    rb = pl.multiple_of(t & ~7, 8)
    colb = pl.multiple_of(step * _SPG, 128)
    return pltpu.make_async_copy(
        prdt_ref.at[pl.ds(rb, 8), pl.ds(colb, 128)],
        buf.at[step % _NBUF, j],
        sems.at[step % _NBUF],
    )

  def step_wait(step):
    pltpu.make_async_copy(
        buf.at[1 - step % _NBUF],
        buf.at[step % _NBUF],
        sems.at[step % _NBUF],
    ).wait()

  @pl.when(g == 0)
  def _():
    acc_ref[...] = jnp.zeros((8, 128), jnp.float32)
    for j in range(_SPG):
      tile_copy(0, j).start()

  @pl.when(g < _G - 1)
  def _():
    for j in range(_SPG):
      tile_copy(g + 1, j).start()

  lane = lax.broadcasted_iota(jnp.int32, (8, 128), 1)
  sub = lax.broadcasted_iota(jnp.int32, (8, 128), 0)
  filled = jnp.ones((8, 128), jnp.float32)
  gb = (g % _NBUF).astype(jnp.int32)
  step_wait(g)
  for j in range(_SPG):
    t = trg_ref[g * _SPG + j]
    filled = jnp.where((sub == (t & 7)) & (lane == j), buf[gb, j], filled)
  acc_ref[...] += jnp.log(filled)

  @pl.when(g == _G - 1)
  def _():
    out_ref[0, 0] = -jnp.sum(acc_ref[...])


@jax.jit
def kernel(prd, trg):
  grid_spec = pltpu.PrefetchScalarGridSpec(
      num_scalar_prefetch=1,
      grid=(_G,),
      in_specs=[pl.BlockSpec(memory_space=pltpu.HBM)],
      out_specs=pl.BlockSpec(
          (1, 1), lambda g, trg_ref: (0, 0), memory_space=pltpu.SMEM
      ),
      scratch_shapes=[
          pltpu.VMEM((_NBUF, _SPG, 8, 128), jnp.float32),
          pltpu.VMEM((8, 128), jnp.float32),
          pltpu.SemaphoreType.DMA((_NBUF,)),
      ],
  )
  out = pl.pallas_call(
      _ce_body,
      grid_spec=grid_spec,
      out_shape=jax.ShapeDtypeStruct((1, 1), jnp.float32),
  )(trg.astype(jnp.int32), prd.T)
  return out[0, 0]
